# baseline (speedup 1.0000x reference)
import functools

import jax
import jax.numpy as jnp
from jax import lax
from jax.experimental import pallas as pl
from jax.experimental.pallas import tpu as pltpu
from jax.experimental.pallas import tpu_sc as plsc

VOCAB = 1000000
EMB = 64
NCLS = 20
L = 200
B = 4096

_info = plsc.get_sparse_core_info()
_NC, _NS = _info.num_cores, _info.num_subcores
_NW = _NC * _NS
_BPW = B // _NW
_VPR = EMB // 16


def _sc_embed_sum(docs32, table):
    mesh = plsc.VectorSubcoreMesh(core_axis_name="c", subcore_axis_name="s")

    @functools.partial(
        pl.kernel,
        mesh=mesh,
        out_type=jax.ShapeDtypeStruct((B, EMB), jnp.float32),
        scratch_types=[
            pltpu.VMEM((L, _BPW), jnp.int32),
            pltpu.VMEM((_BPW, EMB), jnp.float32),
            pltpu.VMEM((_BPW, EMB), jnp.float32),
            pltpu.VMEM((_BPW, EMB), jnp.float32),
            pltpu.SemaphoreType.DMA,
            pltpu.SemaphoreType.DMA,
        ],
        compiler_params=pltpu.CompilerParams(use_tc_tiling_on_sc=False),
    )
    def k(docs_hbm, table_hbm, out_hbm, idx_v, buf0, buf1, acc, sem0, sem1):
        wid = lax.axis_index("s") * _NC + lax.axis_index("c")
        base = wid * _BPW
        pltpu.sync_copy(docs_hbm.at[:, pl.ds(base, _BPW)], idx_v)

        def accum(buf, first):
            def row_body(r, _):
                for c in range(_VPR):
                    s = pl.ds(c * 16, 16)
                    x = buf[r, s]
                    if first:
                        acc[r, s] = x
                    else:
                        plsc.addupdate(acc.at[r, s], x)
                return 0
            lax.fori_loop(0, _BPW, row_body, 0, unroll=4)

        pltpu.async_copy(table_hbm.at[idx_v.at[0]], buf0, sem0)

        def pair_body(lp, _):
            l0 = 2 * lp
            pltpu.make_async_copy(table_hbm.at[idx_v.at[l0]], buf0, sem0).wait()
            pltpu.async_copy(table_hbm.at[idx_v.at[l0 + 1]], buf1, sem1)
            accum(buf0, first=False)
            pltpu.make_async_copy(table_hbm.at[idx_v.at[l0 + 1]], buf1, sem1).wait()

            @pl.when(lp < (L // 2) - 1)
            def _():
                pltpu.async_copy(table_hbm.at[idx_v.at[l0 + 2]], buf0, sem0)

            accum(buf1, first=False)
            return 0

        pltpu.make_async_copy(table_hbm.at[idx_v.at[0]], buf0, sem0).wait()
        pltpu.async_copy(table_hbm.at[idx_v.at[1]], buf1, sem1)
        accum(buf0, first=True)
        pltpu.make_async_copy(table_hbm.at[idx_v.at[1]], buf1, sem1).wait()
        pltpu.async_copy(table_hbm.at[idx_v.at[2]], buf0, sem0)
        accum(buf1, first=False)

        def odd_pair(lp, _):
            return pair_body(lp, _)

        lax.fori_loop(1, L // 2, odd_pair, 0)

        pltpu.sync_copy(acc, out_hbm.at[pl.ds(base, _BPW)])

    return k(docs32, table)


def _tc_classifier(sums, W, b):
    def body(x_ref, w_ref, b_ref, o_ref):
        x = x_ref[...] * (1.0 / L)
        o_ref[...] = (
            lax.dot_general(x, w_ref[...], (((1,), (1,)), ((), ())),
                            preferred_element_type=jnp.float32)
            + b_ref[...]
        )

    return pl.pallas_call(
        body,
        out_shape=jax.ShapeDtypeStruct((B, NCLS), jnp.float32),
    )(sums, W, b.reshape(1, NCLS))


def kernel(docs, table, W, b):
    docs32 = docs.astype(jnp.int32)
    sums = _sc_embed_sum(docs32, table)
    return _tc_classifier(sums, W, b)
